# baseline (speedup 1.0000x reference)
import functools

import jax
import jax.numpy as jnp
from jax import lax
from jax.experimental import pallas as pl
from jax.experimental.pallas import tpu as pltpu
from jax.experimental.pallas import tpu_sc as plsc

T = 2048
V = 100000
NC = 2
NS = 16
NW = NC * NS
BPW = T // NW

_sc_mesh = plsc.VectorSubcoreMesh(core_axis_name="c", subcore_axis_name="s")


@functools.partial(
    pl.kernel,
    out_type=jax.ShapeDtypeStruct((T,), jnp.float32),
    mesh=_sc_mesh,
    scratch_types=[
        pltpu.VMEM((BPW,), jnp.int32),
        pltpu.VMEM((BPW,), jnp.float32),
        pltpu.SemaphoreType.DMA,
    ],
)
def _sc_gather(idx_hbm, table_hbm, out_hbm, idx_v, rows_v, sem):
    wid = lax.axis_index("s") * NC + lax.axis_index("c")
    base = wid * BPW
    pltpu.sync_copy(idx_hbm.at[pl.ds(base, BPW)], idx_v)
    pltpu.async_copy(table_hbm.at[idx_v], rows_v, sem).wait()
    pltpu.sync_copy(rows_v, out_hbm.at[pl.ds(base, BPW)])


_TT = 8
_NBUF = 4
_NSTEP = T // _TT


def _tc_body(x_ref, p_ref, w_ref, b_ref, o_ref, *rest):
    bufs = rest[:_NBUF]
    sems = rest[_NBUF:]

    def outer(o, c):
        for s in range(_NBUF):
            g = o * _NBUF + s

            @pl.when(o > 0)
            def _wait():
                pltpu.make_async_copy(
                    bufs[s], o_ref.at[0, pl.ds(0, _TT), :], sems[s]
                ).wait()

            row0 = pl.multiple_of(g * _TT, _TT)
            xs = x_ref[pl.ds(row0, _TT), :] + p_ref[pl.ds(row0, _TT), :]
            bufs[s][...] = xs * w_ref[...] + b_ref[...]
            pltpu.make_async_copy(
                bufs[s], o_ref.at[0, pl.ds(row0, _TT), :], sems[s]
            ).start(priority=s % 2)
        return c

    lax.fori_loop(0, _NSTEP // _NBUF, outer, 0)
    for s in range(_NBUF):
        pltpu.make_async_copy(bufs[s], o_ref.at[0, pl.ds(0, _TT), :], sems[s]).wait()


def _tc_outer(xg, pos2d, w2d, b2d):
    out = pl.pallas_call(
        _tc_body,
        in_specs=[
            pl.BlockSpec(memory_space=pltpu.MemorySpace.VMEM),
            pl.BlockSpec(memory_space=pltpu.MemorySpace.VMEM),
            pl.BlockSpec(memory_space=pltpu.MemorySpace.VMEM),
            pl.BlockSpec(memory_space=pltpu.MemorySpace.VMEM),
        ],
        out_specs=pl.BlockSpec(memory_space=pl.ANY),
        out_shape=jax.ShapeDtypeStruct((1, T, V), jnp.float32),
        scratch_shapes=(
            [pltpu.VMEM((_TT, V), jnp.float32) for _ in range(_NBUF)]
            + [pltpu.SemaphoreType.DMA for _ in range(_NBUF)]
        ),
    )(xg, pos2d, w2d, b2d)
    return out


def kernel(idx, tok_table, pos_emb, W, b):
    idx1 = idx.reshape(T).astype(jnp.int32)
    table1 = tok_table.reshape(V)
    xg = _sc_gather(idx1, table1)
    return _tc_outer(
        xg.reshape(T, 1),
        pos_emb.reshape(T, 1),
        W.reshape(1, V),
        b.reshape(1, V),
    )


# speedup vs baseline: 3.5611x; 3.5611x over previous
import functools

import jax
import jax.numpy as jnp
from jax import lax
from jax.experimental import pallas as pl
from jax.experimental.pallas import tpu as pltpu
from jax.experimental.pallas import tpu_sc as plsc

T = 2048
V = 100000
NC = 2
NS = 16
NW = NC * NS
BPW = T // NW

_sc_mesh = plsc.VectorSubcoreMesh(core_axis_name="c", subcore_axis_name="s")


@functools.partial(
    pl.kernel,
    out_type=jax.ShapeDtypeStruct((T,), jnp.float32),
    mesh=_sc_mesh,
    scratch_types=[
        pltpu.VMEM((BPW,), jnp.int32),
        pltpu.VMEM((BPW,), jnp.float32),
        pltpu.SemaphoreType.DMA,
    ],
)
def _sc_gather(idx_hbm, table_hbm, out_hbm, idx_v, rows_v, sem):
    wid = lax.axis_index("s") * NC + lax.axis_index("c")
    base = wid * BPW
    pltpu.sync_copy(idx_hbm.at[pl.ds(base, BPW)], idx_v)
    pltpu.async_copy(table_hbm.at[idx_v], rows_v, sem).wait()
    pltpu.sync_copy(rows_v, out_hbm.at[pl.ds(base, BPW)])


_VT = 1024
_NBUF = 4
_NFORI = 96
_VLAST = V - 97 * _VT


def _tc_body(x_ref, p_ref, w_ref, b_ref, o_ref, *rest):
    bufs = rest[:_NBUF]
    sems = rest[_NBUF:]
    xs = x_ref[...] + p_ref[...]

    def _compute(s, row0, nrows):
        ws = w_ref[:, pl.ds(row0, nrows)].reshape(nrows, 1)
        bs = b_ref[:, pl.ds(row0, nrows)].reshape(nrows, 1)
        bufs[s][pl.ds(0, nrows), :] = ws * xs + bs
        pltpu.make_async_copy(
            bufs[s].at[pl.ds(0, nrows), :],
            o_ref.at[0, pl.ds(row0, nrows), :],
            sems[s],
        ).start(priority=s % 2)

    def outer(o, c):
        for s in range(_NBUF):
            g = o * _NBUF + s

            @pl.when(o > 0)
            def _wait():
                pltpu.make_async_copy(
                    bufs[s], o_ref.at[0, pl.ds(0, _VT), :], sems[s]
                ).wait()

            _compute(s, pl.multiple_of(g * _VT, _VT), _VT)
        return c

    lax.fori_loop(0, _NFORI // _NBUF, outer, 0)
    pltpu.make_async_copy(bufs[0], o_ref.at[0, pl.ds(0, _VT), :], sems[0]).wait()
    _compute(0, 96 * _VT, _VT)
    pltpu.make_async_copy(bufs[1], o_ref.at[0, pl.ds(0, _VT), :], sems[1]).wait()
    _compute(1, 97 * _VT, _VLAST)
    pltpu.make_async_copy(bufs[0], o_ref.at[0, pl.ds(0, _VT), :], sems[0]).wait()
    pltpu.make_async_copy(
        bufs[1].at[pl.ds(0, _VLAST), :], o_ref.at[0, pl.ds(0, _VLAST), :], sems[1]
    ).wait()
    pltpu.make_async_copy(bufs[2], o_ref.at[0, pl.ds(0, _VT), :], sems[2]).wait()
    pltpu.make_async_copy(bufs[3], o_ref.at[0, pl.ds(0, _VT), :], sems[3]).wait()


def _tc_outer(x1, p1, w2, b2):
    out = pl.pallas_call(
        _tc_body,
        in_specs=[
            pl.BlockSpec(memory_space=pltpu.MemorySpace.VMEM),
            pl.BlockSpec(memory_space=pltpu.MemorySpace.VMEM),
            pl.BlockSpec(memory_space=pltpu.MemorySpace.VMEM),
            pl.BlockSpec(memory_space=pltpu.MemorySpace.VMEM),
        ],
        out_specs=pl.BlockSpec(memory_space=pl.ANY),
        out_shape=jax.ShapeDtypeStruct((1, V, T), jnp.float32),
        scratch_shapes=(
            [pltpu.VMEM((_VT, T), jnp.float32) for _ in range(_NBUF)]
            + [pltpu.SemaphoreType.DMA for _ in range(_NBUF)]
        ),
    )(x1, p1, w2, b2)
    return jnp.transpose(out, (0, 2, 1))


def kernel(idx, tok_table, pos_emb, W, b):
    idx1 = idx.reshape(T).astype(jnp.int32)
    table1 = tok_table.reshape(V)
    xg = _sc_gather(idx1, table1)
    return _tc_outer(
        xg.reshape(1, T),
        pos_emb.reshape(1, T),
        W.reshape(1, V),
        b.reshape(1, V),
    )


# speedup vs baseline: 3.5622x; 1.0003x over previous
import functools

import jax
import jax.numpy as jnp
from jax import lax
from jax.experimental import pallas as pl
from jax.experimental.pallas import tpu as pltpu
from jax.experimental.pallas import tpu_sc as plsc

T = 2048
V = 100000
NC = 2
NS = 16
NW = NC * NS
BPW = T // NW

_sc_mesh = plsc.VectorSubcoreMesh(core_axis_name="c", subcore_axis_name="s")


@functools.partial(
    pl.kernel,
    out_type=jax.ShapeDtypeStruct((T,), jnp.float32),
    mesh=_sc_mesh,
    scratch_types=[
        pltpu.VMEM((BPW,), jnp.int32),
        pltpu.VMEM((BPW,), jnp.float32),
        pltpu.SemaphoreType.DMA,
    ],
)
def _sc_gather(idx_hbm, table_hbm, out_hbm, idx_v, rows_v, sem):
    wid = lax.axis_index("s") * NC + lax.axis_index("c")
    base = wid * BPW
    pltpu.sync_copy(idx_hbm.at[pl.ds(base, BPW)], idx_v)
    pltpu.async_copy(table_hbm.at[idx_v], rows_v, sem).wait()
    pltpu.sync_copy(rows_v, out_hbm.at[pl.ds(base, BPW)])


_VT = 1024
_NBUF = 4
_NFORI = 96
_VLAST = V - 97 * _VT


def _tc_body(x_ref, p_ref, w_ref, b_ref, o_ref, *rest):
    bufs = rest[:_NBUF]
    sems = rest[_NBUF:]
    xs = x_ref[...] + p_ref[...]

    def _compute(s, row0, nrows):
        ws = w_ref[:, pl.ds(row0, nrows)].reshape(nrows, 1)
        bs = b_ref[pl.ds(row0, nrows)].reshape(nrows, 1)
        bufs[s][pl.ds(0, nrows), :] = ws * xs + bs
        pltpu.make_async_copy(
            bufs[s].at[pl.ds(0, nrows), :],
            o_ref.at[0, pl.ds(row0, nrows), :],
            sems[s],
        ).start(priority=s % 2)

    def outer(o, c):
        for s in range(_NBUF):
            g = o * _NBUF + s

            @pl.when(o > 0)
            def _wait():
                pltpu.make_async_copy(
                    bufs[s], o_ref.at[0, pl.ds(0, _VT), :], sems[s]
                ).wait()

            _compute(s, pl.multiple_of(g * _VT, _VT), _VT)
        return c

    lax.fori_loop(0, _NFORI // _NBUF, outer, 0)
    pltpu.make_async_copy(bufs[0], o_ref.at[0, pl.ds(0, _VT), :], sems[0]).wait()
    _compute(0, 96 * _VT, _VT)
    pltpu.make_async_copy(bufs[1], o_ref.at[0, pl.ds(0, _VT), :], sems[1]).wait()
    _compute(1, 97 * _VT, _VLAST)
    pltpu.make_async_copy(bufs[0], o_ref.at[0, pl.ds(0, _VT), :], sems[0]).wait()
    pltpu.make_async_copy(
        bufs[1].at[pl.ds(0, _VLAST), :], o_ref.at[0, pl.ds(0, _VLAST), :], sems[1]
    ).wait()
    pltpu.make_async_copy(bufs[2], o_ref.at[0, pl.ds(0, _VT), :], sems[2]).wait()
    pltpu.make_async_copy(bufs[3], o_ref.at[0, pl.ds(0, _VT), :], sems[3]).wait()


def _tc_outer(x1, p1, w2, b2):
    out = pl.pallas_call(
        _tc_body,
        in_specs=[
            pl.BlockSpec(memory_space=pltpu.MemorySpace.VMEM),
            pl.BlockSpec(memory_space=pltpu.MemorySpace.VMEM),
            pl.BlockSpec(memory_space=pltpu.MemorySpace.VMEM),
            pl.BlockSpec(memory_space=pltpu.MemorySpace.VMEM),
        ],
        out_specs=pl.BlockSpec(memory_space=pl.ANY),
        out_shape=jax.ShapeDtypeStruct((1, V, T), jnp.float32),
        scratch_shapes=(
            [pltpu.VMEM((_VT, T), jnp.float32) for _ in range(_NBUF)]
            + [pltpu.SemaphoreType.DMA for _ in range(_NBUF)]
        ),
    )(x1, p1, w2, b2)
    return jnp.transpose(out, (0, 2, 1))


def kernel(idx, tok_table, pos_emb, W, b):
    idx1 = idx.reshape(T).astype(jnp.int32)
    table1 = tok_table.reshape(V)
    xg = _sc_gather(idx1, table1)
    return _tc_outer(
        xg.reshape(1, T),
        pos_emb.reshape(1, T),
        W.reshape(1, V),
        b,
    )
